# baseline (speedup 1.0000x reference)
import functools

import jax
import jax.numpy as jnp
from jax import lax
from jax.experimental import pallas as pl
from jax.experimental.pallas import tpu as pltpu
from jax.experimental.pallas import tpu_sc as plsc

_EPS = 1e-6


def _dist_body(B, mu_ref, y_ref, pit_ref, idxa_ref, idxf_ref, cls_ref):
    K, T2, Bb = mu_ref.shape
    T = T2 // 2
    mu = mu_ref[...]
    yt = y_ref[...]
    d = mu - yt[None]
    dists = []
    for t in range(T):
        dx = d[:, 2 * t, :]
        dy = d[:, 2 * t + 1, :]
        dists.append(jnp.sqrt(dx * dx + dy * dy))
    l2 = dists[0]
    for t in range(1, T):
        l2 = l2 + dists[t]
    dfde = dists[T - 1]

    kio = lax.broadcasted_iota(jnp.int32, (K, Bb), 0)
    minv = jnp.min(l2, axis=0)
    best = jnp.min(jnp.where(l2 == minv[None], kio, K), axis=0)
    minf = jnp.min(dfde, axis=0)
    bestf = jnp.min(jnp.where(dfde == minf[None], kio, K), axis=0)

    i = pl.program_id(0)
    boff = (i * Bb + lax.broadcasted_iota(jnp.int32, (1, Bb), 1)) * T2
    jio = lax.broadcasted_iota(jnp.int32, (T2, Bb), 0)
    idxa_ref[...] = jnp.reshape(best, (1, Bb)) * (B * T2) + boff + jio
    idxf_ref[...] = jnp.reshape(bestf, (1, Bb)) * (B * T2) + boff + jio

    z = l2 * (-1.0 / T)
    zm = jnp.max(z, axis=0)
    ez = jnp.exp(z - zm[None])
    st = ez / jnp.sum(ez, axis=0)[None]
    pit = pit_ref[...]
    pm = jnp.max(pit, axis=0)
    lse = jnp.log(jnp.sum(jnp.exp(pit - pm[None]), axis=0)) + pm
    ce = jnp.sum(st * (lse[None] - pit), axis=0)
    cls_part = jnp.sum(ce)

    @pl.when(i == 0)
    def _init():
        cls_ref[...] = jnp.zeros_like(cls_ref)

    cls_ref[...] = cls_ref[...] + jnp.reshape(cls_part, (1, 1))


def _run_dist(mu_t, y_t, pit, K, B, T2, Bb, interpret=False):
    return pl.pallas_call(
        functools.partial(_dist_body, B),
        grid=(B // Bb,),
        in_specs=[
            pl.BlockSpec((K, T2, Bb), lambda i: (0, 0, i)),
            pl.BlockSpec((T2, Bb), lambda i: (0, i)),
            pl.BlockSpec((K, Bb), lambda i: (0, i)),
        ],
        out_specs=[
            pl.BlockSpec((T2, Bb), lambda i: (0, i)),
            pl.BlockSpec((T2, Bb), lambda i: (0, i)),
            pl.BlockSpec((1, 1), lambda i: (0, 0)),
        ],
        out_shape=[
            jax.ShapeDtypeStruct((T2, B), jnp.int32),
            jax.ShapeDtypeStruct((T2, B), jnp.int32),
            jax.ShapeDtypeStruct((1, 1), jnp.float32),
        ],
        interpret=interpret,
    )(mu_t, y_t, pit)


def _gather_sel(mu1, sg1, ia, iff):
    N = ia.shape[0]
    info = plsc.get_sparse_core_info()
    nw = info.num_cores * info.num_subcores
    chunk = N // nw
    mesh = plsc.VectorSubcoreMesh(core_axis_name="c", subcore_axis_name="s")

    @functools.partial(
        pl.kernel, mesh=mesh,
        out_type=[
            jax.ShapeDtypeStruct((N,), jnp.float32),
            jax.ShapeDtypeStruct((N,), jnp.float32),
            jax.ShapeDtypeStruct((N,), jnp.float32),
        ],
        scratch_types=[
            pltpu.VMEM((chunk,), jnp.int32),
            pltpu.VMEM((chunk,), jnp.float32),
            pltpu.SemaphoreType.DMA,
        ],
        compiler_params=pltpu.CompilerParams(use_tc_tiling_on_sc=False),
    )
    def k(mu_hbm, sg_hbm, ia_hbm, if_hbm, oa_hbm, os_hbm, of_hbm,
          idx_v, rows_v, sem):
        wid = lax.axis_index("s") * info.num_cores + lax.axis_index("c")
        base = wid * chunk
        pltpu.sync_copy(ia_hbm.at[pl.ds(base, chunk)], idx_v)
        pltpu.async_copy(mu_hbm.at[idx_v], rows_v, sem).wait()
        pltpu.sync_copy(rows_v, oa_hbm.at[pl.ds(base, chunk)])
        pltpu.async_copy(sg_hbm.at[idx_v], rows_v, sem).wait()
        pltpu.sync_copy(rows_v, os_hbm.at[pl.ds(base, chunk)])
        pltpu.sync_copy(if_hbm.at[pl.ds(base, chunk)], idx_v)
        pltpu.async_copy(mu_hbm.at[idx_v], rows_v, sem).wait()
        pltpu.sync_copy(rows_v, of_hbm.at[pl.ds(base, chunk)])

    return k(mu1, sg1, ia, iff)


def _nll_body(sm_ref, ss_ref, yb_ref, reg_ref):
    sm = sm_ref[...]
    sc = jnp.maximum(ss_ref[...], _EPS)
    nll = jnp.log(2.0 * sc) + jnp.abs(yb_ref[...] - sm) / sc
    reg_ref[...] = jnp.reshape(jnp.sum(nll), (1, 1))


def _run_nll(sm, ss, yb, interpret=False):
    R, C = sm.shape
    return pl.pallas_call(
        _nll_body,
        in_specs=[
            pl.BlockSpec((R, C), lambda: (0, 0)),
            pl.BlockSpec((R, C), lambda: (0, 0)),
            pl.BlockSpec((R, C), lambda: (0, 0)),
        ],
        out_specs=pl.BlockSpec((1, 1), lambda: (0, 0)),
        out_shape=jax.ShapeDtypeStruct((1, 1), jnp.float32),
        interpret=interpret,
    )(sm, ss, yb)


def kernel(out_mu, out_sigma, out_pi, y, pre_obs):
    K, B, T, _ = out_mu.shape
    T2 = 2 * T
    mu2 = out_mu.reshape(K, B, T2)
    mu_t = jnp.transpose(mu2, (0, 2, 1))
    y_t = jnp.transpose(y, (0, 2, 1)).reshape(T2, B)
    pit = jnp.transpose(out_pi, (1, 0))
    Bb = 2048 if B % 2048 == 0 else B

    idxa, idxf, cls = _run_dist(mu_t, y_t, pit, K, B, T2, Bb)

    sel_a, sel_s, sel_f = _gather_sel(
        mu2.reshape(K * B * T2), out_sigma.reshape(K * B * T2),
        idxa.reshape(T2 * B), idxf.reshape(T2 * B))

    flat = (B * T2) // 2048
    reg = _run_nll(sel_a.reshape(flat, 2048), sel_s.reshape(flat, 2048),
                   y_t.reshape(flat, 2048))

    loss = reg[0, 0] / (B * T2) + cls[0, 0] / B
    sk = jnp.transpose(sel_a.reshape(T, 2, B), (0, 2, 1))
    skf = jnp.transpose(sel_f.reshape(T, 2, B), (0, 2, 1))
    tra_ade = jnp.concatenate([pre_obs, sk], axis=0)
    tra_fde = jnp.concatenate([pre_obs, skf], axis=0)
    return (loss, tra_ade, tra_fde)


# speedup vs baseline: 33.0232x; 33.0232x over previous
import jax
import jax.numpy as jnp
from jax import lax
from jax.experimental import pallas as pl

_EPS = 1e-6


def _body(mu_ref, sg_ref, y_ref, pit_ref, sel_ade_ref, sel_fde_ref,
          reg_ref, cls_ref):
    K, T2, Bb = mu_ref.shape
    T = T2 // 2
    mu = mu_ref[...]
    yt = y_ref[...]
    d = mu - yt[None]
    dists = []
    for t in range(T):
        dx = d[:, 2 * t, :]
        dy = d[:, 2 * t + 1, :]
        dists.append(jnp.sqrt(dx * dx + dy * dy))
    l2 = dists[0]
    for t in range(1, T):
        l2 = l2 + dists[t]
    dfde = dists[T - 1]

    kio = lax.broadcasted_iota(jnp.int32, (K, Bb), 0)
    minv = jnp.min(l2, axis=0)
    best = jnp.min(jnp.where(l2 == minv[None], kio, K), axis=0)
    mask = (kio == best[None]).astype(jnp.float32)
    minf = jnp.min(dfde, axis=0)
    bestf = jnp.min(jnp.where(dfde == minf[None], kio, K), axis=0)
    maskf = (kio == bestf[None]).astype(jnp.float32)

    sel_mu = jnp.sum(mask[:, None, :] * mu, axis=0)
    sel_f = jnp.sum(maskf[:, None, :] * mu, axis=0)
    sel_ade_ref[...] = sel_mu
    sel_fde_ref[...] = sel_f

    sg = sg_ref[...]
    sel_sg = jnp.sum(mask[:, :, None] * sg, axis=0)

    absd = jnp.transpose(jnp.abs(yt - sel_mu), (1, 0))
    sc = jnp.maximum(sel_sg, _EPS)
    nll = jnp.log(2.0 * sc) + absd / sc
    reg_part = jnp.sum(nll)

    z = l2 * (-1.0 / T)
    zm = jnp.max(z, axis=0)
    ez = jnp.exp(z - zm[None])
    st = ez / jnp.sum(ez, axis=0)[None]
    pit = pit_ref[...]
    pm = jnp.max(pit, axis=0)
    lse = jnp.log(jnp.sum(jnp.exp(pit - pm[None]), axis=0)) + pm
    ce = jnp.sum(st * (lse[None] - pit), axis=0)
    cls_part = jnp.sum(ce)

    @pl.when(pl.program_id(0) == 0)
    def _init():
        reg_ref[...] = jnp.zeros_like(reg_ref)
        cls_ref[...] = jnp.zeros_like(cls_ref)

    reg_ref[...] = reg_ref[...] + jnp.reshape(reg_part, (1, 1))
    cls_ref[...] = cls_ref[...] + jnp.reshape(cls_part, (1, 1))


def _run(mu_t, sg2, y_t, pit, K, B, T2, Bb, interpret=False):
    return pl.pallas_call(
        _body,
        grid=(B // Bb,),
        in_specs=[
            pl.BlockSpec((K, T2, Bb), lambda i: (0, 0, i)),
            pl.BlockSpec((K, Bb, T2), lambda i: (0, i, 0)),
            pl.BlockSpec((T2, Bb), lambda i: (0, i)),
            pl.BlockSpec((K, Bb), lambda i: (0, i)),
        ],
        out_specs=[
            pl.BlockSpec((T2, Bb), lambda i: (0, i)),
            pl.BlockSpec((T2, Bb), lambda i: (0, i)),
            pl.BlockSpec((1, 1), lambda i: (0, 0)),
            pl.BlockSpec((1, 1), lambda i: (0, 0)),
        ],
        out_shape=[
            jax.ShapeDtypeStruct((T2, B), jnp.float32),
            jax.ShapeDtypeStruct((T2, B), jnp.float32),
            jax.ShapeDtypeStruct((1, 1), jnp.float32),
            jax.ShapeDtypeStruct((1, 1), jnp.float32),
        ],
        interpret=interpret,
    )(mu_t, sg2, y_t, pit)


def kernel(out_mu, out_sigma, out_pi, y, pre_obs):
    K, B, T, _ = out_mu.shape
    T2 = 2 * T
    mu_t = jnp.transpose(out_mu.reshape(K, B, T2), (0, 2, 1))
    sg2 = out_sigma.reshape(K, B, T2)
    y_t = jnp.transpose(y, (0, 2, 1)).reshape(T2, B)
    pit = jnp.transpose(out_pi, (1, 0))
    Bb = 1024 if B % 1024 == 0 else B
    sel_ade, sel_fde, reg, cls = _run(mu_t, sg2, y_t, pit, K, B, T2, Bb)
    loss = reg[0, 0] / (B * T2) + cls[0, 0] / B
    sk = jnp.transpose(sel_ade.reshape(T, 2, B), (0, 2, 1))
    skf = jnp.transpose(sel_fde.reshape(T, 2, B), (0, 2, 1))
    tra_ade = jnp.concatenate([pre_obs, sk], axis=0)
    tra_fde = jnp.concatenate([pre_obs, skf], axis=0)
    return (loss, tra_ade, tra_fde)


# speedup vs baseline: 72.3239x; 2.1901x over previous
import jax
import jax.numpy as jnp
from jax import lax
from jax.experimental import pallas as pl

_EPS = 1e-6


def _body(mu_ref, sg_ref, y_ref, pit_ref, sel_ade_ref, sel_fde_ref,
          reg_ref, cls_ref):
    K, T2, Bb = mu_ref.shape
    T = T2 // 2
    yt = y_ref[...]
    l2 = None
    dfde = None
    for t in range(T):
        dx = mu_ref[:, 2 * t, :] - yt[2 * t][None]
        dy = mu_ref[:, 2 * t + 1, :] - yt[2 * t + 1][None]
        dist = jnp.sqrt(dx * dx + dy * dy)
        l2 = dist if l2 is None else l2 + dist
        if t == T - 1:
            dfde = dist

    kio = lax.broadcasted_iota(jnp.int32, (K, Bb), 0)
    minv = jnp.min(l2, axis=0)
    best = jnp.min(jnp.where(l2 == minv[None], kio, K), axis=0)
    mask = (kio == best[None]).astype(jnp.float32)
    minf = jnp.min(dfde, axis=0)
    bestf = jnp.min(jnp.where(dfde == minf[None], kio, K), axis=0)
    maskf = (kio == bestf[None]).astype(jnp.float32)

    sel_mu = mask[0][None] * mu_ref[0]
    sel_sg = mask[0][None] * sg_ref[0]
    sel_f = maskf[0][None] * mu_ref[0]
    for k in range(1, K):
        sel_mu = sel_mu + mask[k][None] * mu_ref[k]
        sel_sg = sel_sg + mask[k][None] * sg_ref[k]
        sel_f = sel_f + maskf[k][None] * mu_ref[k]
    sel_ade_ref[...] = sel_mu
    sel_fde_ref[...] = sel_f

    sc = jnp.maximum(sel_sg, _EPS)
    nll = jnp.log(2.0 * sc) + jnp.abs(yt - sel_mu) / sc
    reg_part = jnp.sum(nll)

    z = l2 * (-1.0 / T)
    zm = jnp.max(z, axis=0)
    ez = jnp.exp(z - zm[None])
    st = ez / jnp.sum(ez, axis=0)[None]
    pit = pit_ref[...]
    pm = jnp.max(pit, axis=0)
    lse = jnp.log(jnp.sum(jnp.exp(pit - pm[None]), axis=0)) + pm
    ce = jnp.sum(st * (lse[None] - pit), axis=0)
    cls_part = jnp.sum(ce)

    @pl.when(pl.program_id(0) == 0)
    def _init():
        reg_ref[...] = jnp.zeros_like(reg_ref)
        cls_ref[...] = jnp.zeros_like(cls_ref)

    reg_ref[...] = reg_ref[...] + jnp.reshape(reg_part, (1, 1))
    cls_ref[...] = cls_ref[...] + jnp.reshape(cls_part, (1, 1))


def _run(mu_t, sg_t, y_t, pit, K, B, T2, Bb, interpret=False):
    return pl.pallas_call(
        _body,
        grid=(B // Bb,),
        in_specs=[
            pl.BlockSpec((K, T2, Bb), lambda i: (0, 0, i)),
            pl.BlockSpec((K, T2, Bb), lambda i: (0, 0, i)),
            pl.BlockSpec((T2, Bb), lambda i: (0, i)),
            pl.BlockSpec((K, Bb), lambda i: (0, i)),
        ],
        out_specs=[
            pl.BlockSpec((T2, Bb), lambda i: (0, i)),
            pl.BlockSpec((T2, Bb), lambda i: (0, i)),
            pl.BlockSpec((1, 1), lambda i: (0, 0)),
            pl.BlockSpec((1, 1), lambda i: (0, 0)),
        ],
        out_shape=[
            jax.ShapeDtypeStruct((T2, B), jnp.float32),
            jax.ShapeDtypeStruct((T2, B), jnp.float32),
            jax.ShapeDtypeStruct((1, 1), jnp.float32),
            jax.ShapeDtypeStruct((1, 1), jnp.float32),
        ],
        interpret=interpret,
    )(mu_t, sg_t, y_t, pit)


def kernel(out_mu, out_sigma, out_pi, y, pre_obs):
    K, B, T, _ = out_mu.shape
    T2 = 2 * T
    mu_t = jnp.transpose(out_mu.reshape(K, B, T2), (0, 2, 1))
    sg_t = jnp.transpose(out_sigma.reshape(K, B, T2), (0, 2, 1))
    y_t = jnp.transpose(y, (0, 2, 1)).reshape(T2, B)
    pit = jnp.transpose(out_pi, (1, 0))
    Bb = 2048 if B % 2048 == 0 else B
    sel_ade, sel_fde, reg, cls = _run(mu_t, sg_t, y_t, pit, K, B, T2, Bb)
    loss = reg[0, 0] / (B * T2) + cls[0, 0] / B
    sk = jnp.transpose(sel_ade.reshape(T, 2, B), (0, 2, 1))
    skf = jnp.transpose(sel_fde.reshape(T, 2, B), (0, 2, 1))
    tra_ade = jnp.concatenate([pre_obs, sk], axis=0)
    tra_fde = jnp.concatenate([pre_obs, skf], axis=0)
    return (loss, tra_ade, tra_fde)
